# Initial kernel scaffold; baseline (speedup 1.0000x reference)
import jax
import jax.numpy as jnp
from jax.experimental import pallas as pl


def kernel(alias_re_inputs, items, mask, f_adjacency_nodes, f_adjacency_weight, b_adjacency_nodes, b_adjacency_weight, emb_w, f_pos, b_pos, W1_w, W1_b, q1, q2, W2_w, W2_b, q3, q4, W3_w, W3_b, W4_w, W4_b, q5, ln_g, ln_b, fc1_w, fc1_b, fc2_w, fc2_b):
    raise NotImplementedError("write your pallas kernel here")


# speedup vs baseline: 1.1999x; 1.1999x over previous
import functools

import jax
import jax.numpy as jnp
from jax import lax
from jax.experimental import pallas as pl
from jax.experimental.pallas import tpu as pltpu
from jax.experimental.pallas import tpu_sc as plsc

B = 1024
L = 20
D = 64
NV = 1001
V = 1024
NBT = 10
GW = 128
BLK = 32
ALPHA = 0.2
KCONST = 12.0

_f32 = jnp.float32
_HI = lax.Precision.HIGHEST


def _dot(a, b):
    return jnp.dot(a, b, preferred_element_type=_f32, precision=_HI)


def _leaky(x):
    return jnp.where(x >= 0, x, ALPHA * x)


def _prep_body(emb_ref, w1r_ref, w2r_ref, w1l_ref, w2l_ref, posf_ref, posb_ref,
               w1b_ref, w2b_ref, w4r_ref, w4b_ref,
               table_ref, biasf_ref, biasb_ref, bias4_ref):
    emb = emb_ref[...]
    zpad = jnp.zeros((V, GW - D), _f32)
    table_ref[0] = jnp.concatenate([emb, zpad], axis=-1)
    table_ref[1] = jnp.concatenate([_dot(emb, w1r_ref[...]), zpad], axis=-1)
    table_ref[2] = jnp.concatenate([_dot(emb, w2r_ref[...]), zpad], axis=-1)

    def branch_bias(pos_ref, wl_ref, wb_ref):
        pw = _dot(pos_ref[0:8, :], wl_ref[...])
        pw2 = pw[0:2, :]
        tiled = jnp.broadcast_to(pw2[:, None, :], (2, 5, D)).reshape(NBT, D)
        tiled = tiled + wb_ref[...]
        return jnp.concatenate([tiled, jnp.zeros((16 - NBT, D), _f32)], axis=0)

    biasf_ref[...] = branch_bias(posf_ref, w1l_ref, w1b_ref)
    biasb_ref[...] = branch_bias(posb_ref, w2l_ref, w2b_ref)

    p4 = _dot(posf_ref[0:24, :], w4r_ref[...])
    b4 = p4[0:L, :] + w4b_ref[...]
    bias4_ref[...] = jnp.concatenate([b4, jnp.zeros((32 - L, D), _f32)], axis=0)


_prep = pl.pallas_call(
    _prep_body,
    out_shape=(
        jax.ShapeDtypeStruct((3, V, GW), _f32),
        jax.ShapeDtypeStruct((16, D), _f32),
        jax.ShapeDtypeStruct((16, D), _f32),
        jax.ShapeDtypeStruct((32, D), _f32),
    ),
)


_WIN = 128


def _sc_gather(table, idx):
    n = idx.shape[0]
    idx2 = idx.reshape(1, n)
    mesh = plsc.VectorSubcoreMesh(core_axis_name="core",
                                  subcore_axis_name="subcore")

    @functools.partial(
        pl.kernel,
        out_type=jax.ShapeDtypeStruct((n, GW), _f32),
        mesh=mesh,
    )
    def k(table_hbm, i_hbm, o_hbm):
        def body(i_vmem, o_vmem):
            pltpu.sync_copy(table_hbm.at[i_vmem.at[0]], o_vmem)

        pltpu.emit_pipeline(
            body,
            grid=(n // _WIN,),
            in_specs=[pl.BlockSpec((1, _WIN), index_map=lambda i: (0, i))],
            out_specs=[pl.BlockSpec((_WIN, GW), index_map=lambda i: (i, 0))],
            core_axis_name=("core", "subcore"),
            dimension_semantics=(pltpu.PARALLEL,),
        )(i_hbm, o_hbm)

    return k(table, idx2)


def _gat(tar, raw_ref, w_ref, bias_ref, qa, qb_row, qb2_row):
    blk_l = BLK * L
    raw = raw_ref[:, 0:D].reshape(blk_l, NBT, D)
    w = w_ref[...]
    w3 = w[:, :, None]
    m3 = (w3 != 0.0).astype(_f32)
    ne = jnp.tanh(raw + bias_ref[0:NBT, :][None]) * m3
    x = tar[:, None, :] * ne
    ap = _dot(x.reshape(blk_l * NBT, D), qa).reshape(blk_l, NBT, D)
    ap = ap + w3 * qb_row[None]
    a = jnp.sum(_leaky(ap) * qb2_row[None], axis=-1)
    a = a + jnp.where(w != 0.0, 0.0, -10000.0)
    a = a - jnp.max(a, axis=-1, keepdims=True)
    e = jnp.exp(a)
    alpha = e / jnp.sum(e, axis=-1, keepdims=True)
    return jnp.sum(alpha[:, :, None] * ne, axis=1)


def _main_body(gf_ref, gb_ref, gt_ref, wf_ref, wb_ref, al_ref,
               bf_ref, bb_ref, b4_ref, mats_ref, misc_ref, fc1b_ref, embt_ref,
               sp_ref, sd_ref):
    blk_l = BLK * L
    tar = gt_ref[:, 0:D]

    nf = _gat(tar, gf_ref, wf_ref, bf_ref,
              mats_ref[0], misc_ref[0:1, :], misc_ref[1:2, :])
    nb = _gat(tar, gb_ref, wb_ref, bb_ref,
              mats_ref[1], misc_ref[2:3, :], misc_ref[3:4, :])
    neig = nf + nb

    gate = jax.nn.sigmoid(_dot(neig, mats_ref[2]) + _dot(tar, mats_ref[3])
                          + misc_ref[4:5, :])
    fin = gate * neig + (1.0 - gate) * tar
    fin3 = fin.reshape(BLK, L, D)

    al = al_ref[...]
    oh = (al[:, :, None]
          == lax.broadcasted_iota(jnp.int32, (BLK, L, L), 2)).astype(_f32)
    af = jnp.sum(oh[:, :, :, None] * fin3[:, None, :, :], axis=2)

    fp = jnp.tanh(_dot(af.reshape(blk_l, D), mats_ref[4]).reshape(BLK, L, D)
                  + b4_ref[0:L, :][None])
    a5 = jnp.sum(_leaky(fp) * misc_ref[5:6, :][None], axis=-1)
    a5 = a5 - jnp.max(a5, axis=-1, keepdims=True)
    e5 = jnp.exp(a5)
    a5 = e5 / jnp.sum(e5, axis=-1, keepdims=True)
    sess = jnp.sum(a5[:, :, None] * fp, axis=1)

    mu = jnp.mean(sess, axis=-1, keepdims=True)
    var = jnp.mean((sess - mu) ** 2, axis=-1, keepdims=True)
    sess = (sess - mu) / jnp.sqrt(var + 1e-5) * misc_ref[6:7, :] \
        + misc_ref[7:8, :]

    embt = embt_ref[...]
    sp = _dot(sess, embt)
    h = jnp.maximum(_dot(mats_ref[5], embt) + fc1b_ref[:, 0:1], 0.0)
    y = _dot(misc_ref[8:9, :], h) + misc_ref[9:10, 0:1]
    s = jax.nn.sigmoid(y)
    sp_ref[...] = sp[:, :NV]
    sd_ref[...] = (sp * s - KCONST * s)[:, :NV]


def _main(gathered, wf2, wb2, alias, biasf, biasb, bias4, mats, misc, fc1bc,
          embt):
    nblk = B // BLK
    off_b = (B * L * NBT) // (BLK * L * NBT)
    off_t = (2 * B * L * NBT) // (BLK * L)
    grid = (nblk,)
    full = lambda shape: pl.BlockSpec(shape, lambda i: (0, 0))
    full3 = lambda shape: pl.BlockSpec(shape, lambda i: (0, 0, 0))
    return pl.pallas_call(
        _main_body,
        grid=grid,
        in_specs=[
            pl.BlockSpec((BLK * L * NBT, GW), lambda i: (i, 0)),
            pl.BlockSpec((BLK * L * NBT, GW), lambda i, o=off_b: (i + o, 0)),
            pl.BlockSpec((BLK * L, GW), lambda i, o=off_t: (i + o, 0)),
            pl.BlockSpec((BLK * L, NBT), lambda i: (i, 0)),
            pl.BlockSpec((BLK * L, NBT), lambda i: (i, 0)),
            pl.BlockSpec((BLK, L), lambda i: (i, 0)),
            full((16, D)),
            full((16, D)),
            full((32, D)),
            full3((6, D, D)),
            full((16, D)),
            full((D, 8)),
            full((D, V)),
        ],
        out_specs=(
            pl.BlockSpec((BLK, NV), lambda i: (i, 0)),
            pl.BlockSpec((BLK, NV), lambda i: (i, 0)),
        ),
        out_shape=(
            jax.ShapeDtypeStruct((B, NV), _f32),
            jax.ShapeDtypeStruct((B, NV), _f32),
        ),
    )(gathered, gathered, gathered, wf2, wb2, alias, biasf, biasb, bias4,
      mats, misc, fc1bc, embt)


def kernel(alias_re_inputs, items, mask, f_adjacency_nodes, f_adjacency_weight,
           b_adjacency_nodes, b_adjacency_weight, emb_w, f_pos, b_pos,
           W1_w, W1_b, q1, q2, W2_w, W2_b, q3, q4, W3_w, W3_b,
           W4_w, W4_b, q5, ln_g, ln_b, fc1_w, fc1_b, fc2_w, fc2_b):
    emb_pad = jnp.zeros((V, D), _f32).at[1:NV].set(emb_w)

    table, biasf, biasb, bias4 = _prep(
        emb_pad, W1_w[:, D:].T, W2_w[:, D:].T, W1_w[:, :D].T, W2_w[:, :D].T,
        f_pos, b_pos, W1_b.reshape(1, D), W2_b.reshape(1, D),
        W4_w[:, D:].T, W4_b.reshape(1, D))

    idx_f = f_adjacency_nodes.reshape(-1).astype(jnp.int32) + V
    idx_b = b_adjacency_nodes.reshape(-1).astype(jnp.int32) + 2 * V
    idx_t = items.reshape(-1).astype(jnp.int32)
    idx_all = jnp.concatenate([idx_f, idx_b, idx_t])

    gathered = _sc_gather(table.reshape(3 * V, GW), idx_all)

    wf2 = f_adjacency_weight.reshape(B * L, NBT)
    wb2 = b_adjacency_weight.reshape(B * L, NBT)

    mats = jnp.stack([q1[:D], q3[:D], W3_w[:, :D].T, W3_w[:, D:].T,
                      W4_w[:, :D].T, fc1_w])
    misc = jnp.concatenate([
        q1[D:], q2.T, q3[D:], q4.T, W3_b[None], q5.T,
        ln_g[None], ln_b[None], fc2_w, jnp.full((1, D), fc2_w.dtype.type(0)) + fc2_b[0],
    ], axis=0)
    misc = jnp.pad(misc, ((0, 6), (0, 0)))
    fc1bc = jnp.broadcast_to(fc1_b.reshape(D, 1), (D, 8))
    embt = emb_pad.T

    return _main(gathered, wf2, wb2, alias_re_inputs.astype(jnp.int32),
                 biasf, biasb, bias4, mats, misc, fc1bc, embt)


# speedup vs baseline: 1.2355x; 1.0296x over previous
import functools

import jax
import jax.numpy as jnp
from jax import lax
from jax.experimental import pallas as pl
from jax.experimental.pallas import tpu as pltpu
from jax.experimental.pallas import tpu_sc as plsc

B = 1024
L = 20
D = 64
NV = 1001
V = 1024
NBT = 10
GW = 128
BLK = 32
ALPHA = 0.2
KCONST = 12.0

_f32 = jnp.float32
_HI = lax.Precision.HIGHEST


def _dot(a, b):
    return jnp.dot(a, b, preferred_element_type=_f32, precision=_HI)


def _leaky(x):
    return jnp.where(x >= 0, x, ALPHA * x)


def _prep_body(emb_ref, w1r_ref, w2r_ref, w1l_ref, w2l_ref, posf_ref, posb_ref,
               w1b_ref, w2b_ref, w4r_ref, w4b_ref,
               table_ref, bias4_ref):
    emb = emb_ref[...]
    zpad = jnp.zeros((V, GW - D), _f32)
    table_ref[0] = jnp.concatenate([emb, zpad], axis=-1)

    def branch_tables(wr_ref, pos_ref, wl_ref, wb_ref, slot):
        t = _dot(emb, wr_ref[...])
        pw = _dot(pos_ref[0:8, :], wl_ref[...])
        b0 = pw[0:1, :] + wb_ref[...]
        b1 = pw[1:2, :] + wb_ref[...]
        table_ref[slot] = jnp.concatenate([jnp.tanh(t + b0), zpad], axis=-1)
        table_ref[slot + 1] = jnp.concatenate([jnp.tanh(t + b1), zpad], axis=-1)

    branch_tables(w1r_ref, posf_ref, w1l_ref, w1b_ref, 1)
    branch_tables(w2r_ref, posb_ref, w2l_ref, w2b_ref, 3)

    p4 = _dot(posf_ref[0:24, :], w4r_ref[...])
    b4 = p4[0:L, :] + w4b_ref[...]
    bias4_ref[...] = jnp.concatenate([b4, jnp.zeros((32 - L, D), _f32)], axis=0)


_prep = pl.pallas_call(
    _prep_body,
    out_shape=(
        jax.ShapeDtypeStruct((5, V, GW), _f32),
        jax.ShapeDtypeStruct((32, D), _f32),
    ),
)


_WIN = 128


def _sc_gather(table, idx):
    n = idx.shape[0]
    idx2 = idx.reshape(1, n)
    mesh = plsc.VectorSubcoreMesh(core_axis_name="core",
                                  subcore_axis_name="subcore")

    @functools.partial(
        pl.kernel,
        out_type=jax.ShapeDtypeStruct((n, GW), _f32),
        mesh=mesh,
    )
    def k(table_hbm, i_hbm, o_hbm):
        def body(i_vmem, o_vmem):
            pltpu.sync_copy(table_hbm.at[i_vmem.at[0]], o_vmem)

        pltpu.emit_pipeline(
            body,
            grid=(n // _WIN,),
            in_specs=[pl.BlockSpec((1, _WIN), index_map=lambda i: (0, i))],
            out_specs=[pl.BlockSpec((_WIN, GW), index_map=lambda i: (i, 0))],
            core_axis_name=("core", "subcore"),
            dimension_semantics=(pltpu.PARALLEL,),
        )(i_hbm, o_hbm)

    return k(table, idx2)


def _gat(tar, raw_ref, w_ref, qa, qb_row, qb2_row):
    blk_l = BLK * L
    raw = raw_ref[:, 0:D].reshape(blk_l, NBT, D)
    w = w_ref[...]
    w3 = w[:, :, None]
    m3 = (w3 != 0.0).astype(_f32)
    ne = raw * m3
    x = tar[:, None, :] * ne
    ap = _dot(x.reshape(blk_l * NBT, D), qa).reshape(blk_l, NBT, D)
    ap = ap + w3 * qb_row[None]
    a = jnp.sum(_leaky(ap) * qb2_row[None], axis=-1)
    a = a + jnp.where(w != 0.0, 0.0, -10000.0)
    a = a - jnp.max(a, axis=-1, keepdims=True)
    e = jnp.exp(a)
    alpha = e / jnp.sum(e, axis=-1, keepdims=True)
    return jnp.sum(alpha[:, :, None] * ne, axis=1)


def _main_body(gf_ref, gb_ref, gt_ref, wf_ref, wb_ref, al_ref,
               b4_ref, mats_ref, misc_ref, fc1b_ref, embt_ref,
               sp_ref, sd_ref):
    blk_l = BLK * L
    tar = gt_ref[:, 0:D]

    nf = _gat(tar, gf_ref, wf_ref,
              mats_ref[0], misc_ref[0:1, :], misc_ref[1:2, :])
    nb = _gat(tar, gb_ref, wb_ref,
              mats_ref[1], misc_ref[2:3, :], misc_ref[3:4, :])
    neig = nf + nb

    gate = jax.nn.sigmoid(_dot(neig, mats_ref[2]) + _dot(tar, mats_ref[3])
                          + misc_ref[4:5, :])
    fin = gate * neig + (1.0 - gate) * tar
    fin3 = fin.reshape(BLK, L, D)

    al = al_ref[...]
    oh = (al[:, :, None]
          == lax.broadcasted_iota(jnp.int32, (BLK, L, L), 2)).astype(_f32)
    af = jnp.sum(oh[:, :, :, None] * fin3[:, None, :, :], axis=2)

    fp = jnp.tanh(_dot(af.reshape(blk_l, D), mats_ref[4]).reshape(BLK, L, D)
                  + b4_ref[0:L, :][None])
    a5 = jnp.sum(_leaky(fp) * misc_ref[5:6, :][None], axis=-1)
    a5 = a5 - jnp.max(a5, axis=-1, keepdims=True)
    e5 = jnp.exp(a5)
    a5 = e5 / jnp.sum(e5, axis=-1, keepdims=True)
    sess = jnp.sum(a5[:, :, None] * fp, axis=1)

    mu = jnp.mean(sess, axis=-1, keepdims=True)
    var = jnp.mean((sess - mu) ** 2, axis=-1, keepdims=True)
    sess = (sess - mu) / jnp.sqrt(var + 1e-5) * misc_ref[6:7, :] \
        + misc_ref[7:8, :]

    embt = embt_ref[...]
    sp = _dot(sess, embt)
    h = jnp.maximum(_dot(mats_ref[5], embt) + fc1b_ref[:, 0:1], 0.0)
    y = _dot(misc_ref[8:9, :], h) + misc_ref[9:10, 0:1]
    s = jax.nn.sigmoid(y)
    sp_ref[...] = sp[:, :NV]
    sd_ref[...] = (sp * s - KCONST * s)[:, :NV]


def _main(gathered, wf2, wb2, alias, bias4, mats, misc, fc1bc, embt):
    nblk = B // BLK
    off_b = (B * L * NBT) // (BLK * L * NBT)
    off_t = (2 * B * L * NBT) // (BLK * L)
    grid = (nblk,)
    full = lambda shape: pl.BlockSpec(shape, lambda i: (0, 0))
    full3 = lambda shape: pl.BlockSpec(shape, lambda i: (0, 0, 0))
    return pl.pallas_call(
        _main_body,
        grid=grid,
        in_specs=[
            pl.BlockSpec((BLK * L * NBT, GW), lambda i: (i, 0)),
            pl.BlockSpec((BLK * L * NBT, GW), lambda i, o=off_b: (i + o, 0)),
            pl.BlockSpec((BLK * L, GW), lambda i, o=off_t: (i + o, 0)),
            pl.BlockSpec((BLK * L, NBT), lambda i: (i, 0)),
            pl.BlockSpec((BLK * L, NBT), lambda i: (i, 0)),
            pl.BlockSpec((BLK, L), lambda i: (i, 0)),
            full((32, D)),
            full3((6, D, D)),
            full((16, D)),
            full((D, 8)),
            full((D, V)),
        ],
        out_specs=(
            pl.BlockSpec((BLK, NV), lambda i: (i, 0)),
            pl.BlockSpec((BLK, NV), lambda i: (i, 0)),
        ),
        out_shape=(
            jax.ShapeDtypeStruct((B, NV), _f32),
            jax.ShapeDtypeStruct((B, NV), _f32),
        ),
    )(gathered, gathered, gathered, wf2, wb2, alias, bias4,
      mats, misc, fc1bc, embt)


def kernel(alias_re_inputs, items, mask, f_adjacency_nodes, f_adjacency_weight,
           b_adjacency_nodes, b_adjacency_weight, emb_w, f_pos, b_pos,
           W1_w, W1_b, q1, q2, W2_w, W2_b, q3, q4, W3_w, W3_b,
           W4_w, W4_b, q5, ln_g, ln_b, fc1_w, fc1_b, fc2_w, fc2_b):
    emb_pad = jnp.zeros((V, D), _f32).at[1:NV].set(emb_w)

    table, bias4 = _prep(
        emb_pad, W1_w[:, D:].T, W2_w[:, D:].T, W1_w[:, :D].T, W2_w[:, :D].T,
        f_pos, b_pos, W1_b.reshape(1, D), W2_b.reshape(1, D),
        W4_w[:, D:].T, W4_b.reshape(1, D))

    so_off = (jnp.arange(2, dtype=jnp.int32) * V).reshape(1, 1, 2, 1)
    idx_f = (f_adjacency_nodes.astype(jnp.int32) + V + so_off).reshape(-1)
    idx_b = (b_adjacency_nodes.astype(jnp.int32) + 3 * V + so_off).reshape(-1)
    idx_t = items.reshape(-1).astype(jnp.int32)
    idx_all = jnp.concatenate([idx_f, idx_b, idx_t])

    gathered = _sc_gather(table.reshape(5 * V, GW), idx_all)

    wf2 = f_adjacency_weight.reshape(B * L, NBT)
    wb2 = b_adjacency_weight.reshape(B * L, NBT)

    mats = jnp.stack([q1[:D], q3[:D], W3_w[:, :D].T, W3_w[:, D:].T,
                      W4_w[:, :D].T, fc1_w])
    misc = jnp.concatenate([
        q1[D:], q2.T, q3[D:], q4.T, W3_b[None], q5.T,
        ln_g[None], ln_b[None], fc2_w, jnp.full((1, D), fc2_w.dtype.type(0)) + fc2_b[0],
    ], axis=0)
    misc = jnp.pad(misc, ((0, 6), (0, 0)))
    fc1bc = jnp.broadcast_to(fc1_b.reshape(D, 1), (D, 8))
    embt = emb_pad.T

    return _main(gathered, wf2, wb2, alias_re_inputs.astype(jnp.int32),
                 bias4, mats, misc, fc1bc, embt)
